# baseline (speedup 1.0000x reference)
import functools

import jax
import jax.numpy as jnp
from jax import lax
from jax.experimental import pallas as pl
from jax.experimental.pallas import tpu as pltpu
from jax.experimental.pallas import tpu_sc as plsc

NC = 2
NS = 16
NW = NC * NS
L = 16
K = 1024
CH = 32768
NBUF = 3
UNROLL = 4


def _col_threshold(tbl_v, tc, c):
    tcb = jnp.full((L,), tc)
    ninf = jnp.full((L,), -jnp.inf, jnp.float32)
    pinf = jnp.full((L,), jnp.inf, jnp.float32)
    lanes = lax.iota(jnp.int32, L)

    def body(j, carry):
        lmax, rmin = carry
        ridx = (lanes + j * L) * 4
        h1 = plsc.load_gather(tbl_v, [ridx])
        hc = plsc.load_gather(tbl_v, [ridx + c])
        below = hc < tcb
        lmax = jnp.maximum(lmax, jnp.where(below, h1, ninf))
        rmin = jnp.minimum(rmin, jnp.where(below, pinf, h1))
        return lmax, rmin

    lmax, rmin = lax.fori_loop(0, K // L, body, (ninf, pinf))
    return 0.5 * (jnp.max(lmax) + jnp.min(rmin))


def _sc_body(n, x_hbm, hf_hbm, t_hbm, d_hbm, b_hbm, out_hbm,
             buf0, buf1, buf2, tbl_v, tv_v, dv_v, bv_v,
             isem0, isem1, isem2, osem0, osem1, osem2):
    wid = lax.axis_index("s") * NC + lax.axis_index("c")
    per = n // NW
    base = wid * per

    bufs = (buf0, buf1, buf2)
    isems = (isem0, isem1, isem2)
    osems = (osem0, osem1, osem2)
    nch = per // CH
    in_d = [None] * nch
    out_d = [None] * nch

    def start_in(ch):
        s = ch % NBUF
        in_d[ch] = pltpu.async_copy(
            x_hbm.at[pl.ds(base + ch * CH, CH)], bufs[s], isems[s])

    start_in(0)
    start_in(1)
    start_in(2)

    p1 = pltpu.async_copy(hf_hbm, tbl_v, osem0)
    p2 = pltpu.async_copy(t_hbm, tv_v.at[pl.ds(0, 4)], osem1)
    p3 = pltpu.async_copy(d_hbm, dv_v.at[pl.ds(0, 4)], osem2)
    p4 = pltpu.async_copy(b_hbm, bv_v.at[pl.ds(0, 1)], isem2)
    p1.wait()
    p2.wait()
    p3.wait()
    p4.wait()

    tvec = tv_v[...]
    dvec = dv_v[...]
    b = bv_v[...][0]
    pairs = [
        (_col_threshold(tbl_v, tvec[0], 0), dvec[0]),
        (tvec[1], dvec[1]),
        (_col_threshold(tbl_v, tvec[2], 2), dvec[2]),
        (_col_threshold(tbl_v, tvec[3], 3), dvec[3]),
    ]

    def cswap(i, j):
        ti, di = pairs[i]
        tj, dj = pairs[j]
        m = ti <= tj
        pairs[i] = (jnp.where(m, ti, tj), jnp.where(m, di, dj))
        pairs[j] = (jnp.where(m, tj, ti), jnp.where(m, dj, di))

    for i, j in ((0, 1), (2, 3), (0, 2), (1, 3), (1, 2)):
        cswap(i, j)

    s = -b
    sv = [jnp.full((L,), s)]
    for _, dc in pairs:
        s = s + dc
        sv.append(jnp.full((L,), s))
    tv = [jnp.full((L,), tc) for tc, _ in pairs]

    def compute(buf):
        @pl.loop(0, CH, step=L, unroll=UNROLL)
        def _compute(i):
            xv = buf[pl.ds(i, L)]
            hi = jnp.where(xv >= tv[3], sv[4], sv[3])
            hi = jnp.where(xv >= tv[2], hi, sv[2])
            lo = jnp.where(xv >= tv[0], sv[1], sv[0])
            buf[pl.ds(i, L)] = jnp.where(xv >= tv[1], hi, lo)

    for ch in range(nch):
        s = ch % NBUF
        if 2 <= ch and ch + 1 < nch:
            if ch + 1 - NBUF >= 0:
                out_d[ch + 1 - NBUF].wait()
            start_in(ch + 1)
        in_d[ch].wait()
        compute(bufs[s])
        out_d[ch] = pltpu.async_copy(
            bufs[s], out_hbm.at[pl.ds(base + ch * CH, CH)], osems[s])
    for ch in range(nch - NBUF, nch):
        out_d[ch].wait()


def kernel(x, h, d, T, b):
    n = x.shape[0]
    assert n % (NW * CH) == 0

    hf = h.reshape(K * 4)
    b1 = jnp.reshape(b, (1,))

    mesh = plsc.VectorSubcoreMesh(
        core_axis_name="c", subcore_axis_name="s",
        num_cores=NC, num_subcores=NS)
    run = pl.kernel(
        functools.partial(_sc_body, n),
        out_type=jax.ShapeDtypeStruct((n,), jnp.float32),
        mesh=mesh,
        compiler_params=pltpu.CompilerParams(needs_layout_passes=False),
        scratch_types=[
            pltpu.VMEM((CH,), jnp.float32),
            pltpu.VMEM((CH,), jnp.float32),
            pltpu.VMEM((CH,), jnp.float32),
            pltpu.VMEM((K * 4,), jnp.float32),
            pltpu.VMEM((L,), jnp.float32),
            pltpu.VMEM((L,), jnp.float32),
            pltpu.VMEM((L,), jnp.float32),
            pltpu.SemaphoreType.DMA,
            pltpu.SemaphoreType.DMA,
            pltpu.SemaphoreType.DMA,
            pltpu.SemaphoreType.DMA,
            pltpu.SemaphoreType.DMA,
            pltpu.SemaphoreType.DMA,
        ],
    )
    return run(x, hf, T, d, b1)


# speedup vs baseline: 1.0269x; 1.0269x over previous
import functools

import jax
import jax.numpy as jnp
from jax import lax
from jax.experimental import pallas as pl
from jax.experimental.pallas import tpu as pltpu
from jax.experimental.pallas import tpu_sc as plsc

NC = 2
NS = 16
NW = NC * NS
L = 16
K = 1024
CH = 32768
NBUF = 3
UNROLL = 4


def _col_threshold(tbl_v, tc, c):
    tcb = jnp.full((L,), tc)
    ninf = jnp.full((L,), -jnp.inf, jnp.float32)
    pinf = jnp.full((L,), jnp.inf, jnp.float32)
    lanes = lax.iota(jnp.int32, L)

    def body(j, carry):
        lmax, rmin = carry
        ridx = (lanes + j * L) * 4
        h1 = plsc.load_gather(tbl_v, [ridx])
        hc = plsc.load_gather(tbl_v, [ridx + c])
        below = hc < tcb
        lmax = jnp.maximum(lmax, jnp.where(below, h1, ninf))
        rmin = jnp.minimum(rmin, jnp.where(below, pinf, h1))
        return lmax, rmin

    lmax, rmin = lax.fori_loop(0, K // L, body, (ninf, pinf))
    return 0.5 * (jnp.max(lmax) + jnp.min(rmin))


def _sc_body(n, x_hbm, hf_hbm, t_hbm, d_hbm, b_hbm, out_hbm,
             buf0, buf1, buf2, tbl_v, tv_v, dv_v, bv_v,
             isem0, isem1, isem2, osem0, osem1, osem2,
             psem0, psem1, psem2, psem3):
    wid = lax.axis_index("s") * NC + lax.axis_index("c")
    per = n // NW
    base = wid * per

    bufs = (buf0, buf1, buf2)
    isems = (isem0, isem1, isem2)
    osems = (osem0, osem1, osem2)
    nch = per // CH
    in_d = [None] * nch
    out_d = [None] * nch

    def start_in(ch):
        s = ch % NBUF
        in_d[ch] = pltpu.async_copy(
            x_hbm.at[pl.ds(base + ch * CH, CH)], bufs[s], isems[s])

    start_in(0)
    start_in(1)

    p1 = pltpu.async_copy(hf_hbm, tbl_v, psem0)
    p2 = pltpu.async_copy(t_hbm, tv_v.at[pl.ds(0, 4)], psem1)
    p3 = pltpu.async_copy(d_hbm, dv_v.at[pl.ds(0, 4)], psem2)
    p4 = pltpu.async_copy(b_hbm, bv_v.at[pl.ds(0, 1)], psem3)
    p1.wait()
    p2.wait()
    p3.wait()
    p4.wait()

    tvec = tv_v[...]
    dvec = dv_v[...]
    b = bv_v[...][0]
    pairs = [
        (_col_threshold(tbl_v, tvec[0], 0), dvec[0]),
        (tvec[1], dvec[1]),
        (_col_threshold(tbl_v, tvec[2], 2), dvec[2]),
        (_col_threshold(tbl_v, tvec[3], 3), dvec[3]),
    ]

    def cswap(i, j):
        ti, di = pairs[i]
        tj, dj = pairs[j]
        m = ti <= tj
        pairs[i] = (jnp.where(m, ti, tj), jnp.where(m, di, dj))
        pairs[j] = (jnp.where(m, tj, ti), jnp.where(m, dj, di))

    for i, j in ((0, 1), (2, 3), (0, 2), (1, 3), (1, 2)):
        cswap(i, j)

    s = -b
    sv = [jnp.full((L,), s)]
    for _, dc in pairs:
        s = s + dc
        sv.append(jnp.full((L,), s))
    tv = [jnp.full((L,), tc) for tc, _ in pairs]

    def compute(buf):
        @pl.loop(0, CH, step=L, unroll=UNROLL)
        def _compute(i):
            xv = buf[pl.ds(i, L)]
            hi = jnp.where(xv >= tv[3], sv[4], sv[3])
            hi = jnp.where(xv >= tv[2], hi, sv[2])
            lo = jnp.where(xv >= tv[0], sv[1], sv[0])
            buf[pl.ds(i, L)] = jnp.where(xv >= tv[1], hi, lo)

    for ch in range(nch):
        s = ch % NBUF
        if 1 <= ch and ch + 1 < nch:
            if ch + 1 - NBUF >= 0:
                out_d[ch + 1 - NBUF].wait()
            start_in(ch + 1)
        in_d[ch].wait()
        compute(bufs[s])
        out_d[ch] = pltpu.async_copy(
            bufs[s], out_hbm.at[pl.ds(base + ch * CH, CH)], osems[s])
    for ch in range(nch - NBUF, nch):
        out_d[ch].wait()


def kernel(x, h, d, T, b):
    n = x.shape[0]
    assert n % (NW * CH) == 0

    hf = h.reshape(K * 4)
    b1 = jnp.reshape(b, (1,))

    mesh = plsc.VectorSubcoreMesh(
        core_axis_name="c", subcore_axis_name="s",
        num_cores=NC, num_subcores=NS)
    run = pl.kernel(
        functools.partial(_sc_body, n),
        out_type=jax.ShapeDtypeStruct((n,), jnp.float32),
        mesh=mesh,
        compiler_params=pltpu.CompilerParams(needs_layout_passes=False),
        scratch_types=[
            pltpu.VMEM((CH,), jnp.float32),
            pltpu.VMEM((CH,), jnp.float32),
            pltpu.VMEM((CH,), jnp.float32),
            pltpu.VMEM((K * 4,), jnp.float32),
            pltpu.VMEM((L,), jnp.float32),
            pltpu.VMEM((L,), jnp.float32),
            pltpu.VMEM((L,), jnp.float32),
            pltpu.SemaphoreType.DMA,
            pltpu.SemaphoreType.DMA,
            pltpu.SemaphoreType.DMA,
            pltpu.SemaphoreType.DMA,
            pltpu.SemaphoreType.DMA,
            pltpu.SemaphoreType.DMA,
            pltpu.SemaphoreType.DMA,
            pltpu.SemaphoreType.DMA,
            pltpu.SemaphoreType.DMA,
            pltpu.SemaphoreType.DMA,
        ],
    )
    return run(x, hf, T, d, b1)
